# baseline (speedup 1.0000x reference)
import dataclasses

import jax
import jax.numpy as jnp
from jax import lax
from jax.experimental import pallas as pl
from jax.experimental.pallas import tpu as pltpu
from jax.experimental.pallas import tpu_sc as plsc

NC = 2
NS = 16
CHUNK = 32
DEGW = 16


def _sc_aggregate(val, src3d, dst3d, n_nodes, d, with_deg):
  nchunk = src3d.shape[1]
  half = n_nodes // NC
  nacc = half + 8
  zparts = 5
  rpt = half // zparts
  assert half % zparts == 0 and rpt % 8 == 0 and CHUNK % 16 == 0

  mesh = plsc.VectorSubcoreMesh(core_axis_name="c", subcore_axis_name="s")

  out_type = [jax.ShapeDtypeStruct((n_nodes, d), jnp.float32)]
  if with_deg:
    out_type.append(jax.ShapeDtypeStruct((NC, NS, nacc), jnp.float32))

  scratch = [
      pltpu.VMEM((3, CHUNK), jnp.int32),
      pltpu.VMEM((3, CHUNK), jnp.int32),
      pltpu.VMEM((CHUNK, d), jnp.float32),
      pltpu.VMEM((CHUNK, d), jnp.float32),
      pltpu.VMEM((CHUNK, d), jnp.float32),
      pltpu.VMEM_SHARED((nacc, d), jnp.float32),
      pltpu.SemaphoreType.DMA,
      pltpu.SemaphoreType.DMA,
      pltpu.SemaphoreType.DMA,
      pltpu.SemaphoreType.DMA,
      pltpu.SemaphoreType.DMA,
      pltpu.SemaphoreType.DMA,
  ]
  if with_deg:
    scratch.append(pltpu.VMEM((nacc,), jnp.float32))

  def body(val_hbm, src_hbm, dst_hbm, *rest):
    if with_deg:
      (acc_hbm, deg_hbm, si, di, rows0, rows1, rows2, acc, sem0, sem1, sem2,
       isem0, isem1, isem2, histo) = rest
    else:
      (acc_hbm, si, di, rows0, rows1, rows2, acc, sem0, sem1, sem2,
       isem0, isem1, isem2) = rest
    c = lax.axis_index("c")
    s = lax.axis_index("s")
    base = c * half
    z16 = jnp.zeros((16,), jnp.float32)
    one16 = jnp.ones((16,), jnp.float32)

    @pl.loop(0, CHUNK)
    def _(r):
      @pl.loop(0, d // 16)
      def _(u):
        rows0[r, pl.ds(u * 16, 16)] = z16

    if with_deg:
      @pl.loop(0, nacc // 16)
      def _(u):
        histo[pl.ds(u * 16, 16)] = z16

    nz = rpt // CHUNK
    ztail = rpt - nz * CHUNK

    @pl.when(s < zparts)
    def _():
      @pl.loop(0, nz)
      def _(k):
        pltpu.sync_copy(rows0, acc.at[pl.ds(s * rpt + k * CHUNK, CHUNK)])
      if ztail:
        pltpu.sync_copy(rows0.at[pl.ds(0, ztail)],
                        acc.at[pl.ds(s * rpt + nz * CHUNK, ztail)])

    @pl.when(s == zparts)
    def _():
      pltpu.sync_copy(rows0.at[pl.ds(0, 8)], acc.at[pl.ds(half, 8)])

    plsc.subcore_barrier()

    def idx_start(j, p, isem):
      pltpu.make_async_copy(src_hbm.at[s, j], si.at[p], isem).start()
      pltpu.make_async_copy(dst_hbm.at[s, j], di.at[p], isem).start()

    def idx_wait(j, p, isem):
      pltpu.make_async_copy(src_hbm.at[s, j], si.at[p], isem).wait()
      pltpu.make_async_copy(dst_hbm.at[s, j], di.at[p], isem).wait()

    def remap(p):
      @pl.loop(0, CHUNK // 16)
      def _(u):
        v = di[p, pl.ds(u * 16, 16)] - base
        ok = (v >= 0) & (v < half)
        di[p, pl.ds(u * 16, 16)] = jnp.where(ok, v, half)

    def gather_start(j, p, rows, sem):
      pltpu.make_async_copy(val_hbm.at[si.at[p]], rows, sem).start()

    def gather_wait(j, p, rows, sem):
      pltpu.make_async_copy(val_hbm.at[si.at[p]], rows, sem).wait()

    def scatter(p, rows):
      pltpu.sync_copy(rows, acc.at[di.at[p]], add=True)
      if with_deg:
        @pl.loop(0, CHUNK // 16)
        def _(u):
          v = di[p, pl.ds(u * 16, 16)]
          plsc.addupdate_scatter(histo, [v], one16)

    allrows = [rows0, rows1, rows2]
    allsems = [sem0, sem1, sem2]
    allisems = [isem0, isem1, isem2]

    def prep(j, p):
      idx_start(j, p, allisems[p])
      idx_wait(j, p, allisems[p])
      remap(p)
      gather_start(j, p, allrows[p], allsems[p])

    def fin(p):
      gather_wait(0, p, allrows[p], allsems[p])
      scatter(p, allrows[p])

    assert nchunk % 3 == 0
    prep(0, 0)
    prep(1, 1)
    prep(2, 2)

    @pl.loop(0, nchunk - 3, step=3)
    def _(j):
      fin(0)
      prep(j + 3, 0)
      fin(1)
      prep(j + 4, 1)
      fin(2)
      prep(j + 5, 2)

    fin(0)
    fin(1)
    fin(2)

    plsc.subcore_barrier()

    @pl.when(s < zparts)
    def _():
      pltpu.sync_copy(acc.at[pl.ds(s * rpt, rpt)],
                      acc_hbm.at[pl.ds(base + s * rpt, rpt)])
    if with_deg:
      pltpu.sync_copy(histo, deg_hbm.at[c, s])

  cp = pltpu.CompilerParams()
  if "needs_layout_passes" in pltpu.CompilerParams.__dataclass_fields__:
    cp = dataclasses.replace(cp, needs_layout_passes=False)
  k = pl.kernel(body, out_type=out_type, mesh=mesh, scratch_types=scratch,
                compiler_params=cp)
  return k(val, src3d, dst3d)


def _tc_deg_merge(histos, n, half):
  def body(h_ref, o_ref):
    for cc in range(NC):
      dsum = jnp.sum(h_ref[cc, :, :half], axis=0)
      o_ref[cc] = jnp.broadcast_to(dsum[:, None], (half, DEGW))

  out = pl.pallas_call(
      body,
      out_shape=jax.ShapeDtypeStruct((NC, half, DEGW), jnp.float32),
  )(histos)
  return out.reshape(n, DEGW)


def _tc_layer(agg_sum, deg, h_in, wl, bl, wr, gamma=None, beta=None,
              block_rows=1000):
  n, d = h_in.shape
  norm = gamma is not None
  grid = (n // block_rows,)

  def body(a_ref, deg_ref, h_ref, wl_ref, bl_ref, wr_ref, *rest):
    if norm:
      g_ref, b_ref, o_ref = rest
    else:
      (o_ref,) = rest
    degs = jnp.maximum(deg_ref[:, :1], 1.0)
    agg = a_ref[...] / degs
    out = lax.dot_general(agg, wl_ref[...], (((1,), (1,)), ((), ())),
                          preferred_element_type=jnp.float32)
    out = out + lax.dot_general(h_ref[...], wr_ref[...],
                                (((1,), (1,)), ((), ())),
                                preferred_element_type=jnp.float32)
    out = out + bl_ref[...]
    if norm:
      mu = jnp.mean(out, axis=1, keepdims=True)
      var = jnp.mean((out - mu) ** 2, axis=1, keepdims=True)
      out = (out - mu) / jnp.sqrt(var + 1e-5) * g_ref[...] + b_ref[...]
    o_ref[...] = out

  in_specs = [
      pl.BlockSpec((block_rows, d), lambda i: (i, 0)),
      pl.BlockSpec((block_rows, DEGW), lambda i: (i, 0)),
      pl.BlockSpec((block_rows, d), lambda i: (i, 0)),
      pl.BlockSpec((d, d), lambda i: (0, 0)),
      pl.BlockSpec((1, d), lambda i: (0, 0)),
      pl.BlockSpec((d, d), lambda i: (0, 0)),
  ]
  args = [agg_sum, deg, h_in, wl, bl.reshape(1, d), wr]
  if norm:
    in_specs += [pl.BlockSpec((1, d), lambda i: (0, 0)),
                 pl.BlockSpec((1, d), lambda i: (0, 0))]
    args += [gamma.reshape(1, d), beta.reshape(1, d)]

  return pl.pallas_call(
      body,
      grid=grid,
      in_specs=in_specs,
      out_specs=pl.BlockSpec((block_rows, d), lambda i: (i, 0)),
      out_shape=jax.ShapeDtypeStruct((n, d), jnp.float32),
  )(*args)


def kernel(x, edge_index, Wl0, bl0, Wr0, Wl1, bl1, Wr1, gamma, beta):
  n, d = x.shape
  e = edge_index.shape[1]
  assert n % 2 == 0 and d % 16 == 0
  nchunk = -(-e // (NS * CHUNK))
  nchunk += (-nchunk) % 3
  epad = NS * nchunk * CHUNK - e
  src3d = jnp.concatenate([edge_index[0], jnp.zeros((epad,), jnp.int32)]
                          ).reshape(NS, nchunk, CHUNK)
  dst3d = jnp.concatenate([edge_index[1], jnp.full((epad,), -1, jnp.int32)]
                          ).reshape(NS, nchunk, CHUNK)

  s1, histos = _sc_aggregate(x, src3d, dst3d, n, d, with_deg=True)
  deg = _tc_deg_merge(histos, n, n // NC)
  h1 = _tc_layer(s1, deg, x, Wl0, bl0, Wr0)
  (s2,) = _sc_aggregate(h1, src3d, dst3d, n, d, with_deg=False)
  return _tc_layer(s2, deg, h1, Wl1, bl1, Wr1, gamma=gamma, beta=beta)
